# Initial kernel scaffold; baseline (speedup 1.0000x reference)
import jax
import jax.numpy as jnp
from jax.experimental import pallas as pl


def kernel(x, hist_loss, Wp, bp, W_ih, W_hh, b_ih, b_hh, Wfc, bfc):
    raise NotImplementedError("write your pallas kernel here")


# speedup vs baseline: 1.0290x; 1.0290x over previous
import functools

import jax
import jax.numpy as jnp
from jax.experimental import pallas as pl
from jax.experimental.pallas import tpu as pltpu

B, D, S, T, H, HOR = 4096, 256, 16, 220, 64, 20
TP = T - HOR
TAU = 1.0


def _lstm_router_kernel(xs_ref, x_ref, Wcat_ref, b_ref, WpT_ref, WfcT_ref,
                        bfc_ref, gn_ref, final_ref, preds_ref, h_ref, c_ref,
                        hc_ref):
    t = pl.program_id(0)

    @pl.when(t == 0)
    def _init():
        h_ref[...] = jnp.zeros_like(h_ref)
        c_ref[...] = jnp.zeros_like(c_ref)

    x_t = xs_ref[0]
    h = h_ref[...]
    c = c_ref[...]
    hc_ref[:, :S] = x_t
    hc_ref[:, S:] = h
    gates = jnp.dot(hc_ref[...], Wcat_ref[...],
                    preferred_element_type=jnp.float32) + b_ref[...]
    i = jax.nn.sigmoid(gates[:, 0 * H:1 * H])
    f = jax.nn.sigmoid(gates[:, 1 * H:2 * H])
    g = jnp.tanh(gates[:, 2 * H:3 * H])
    o = jax.nn.sigmoid(gates[:, 3 * H:4 * H])
    c = f * c + i * g
    h = o * jnp.tanh(c)
    h_ref[...] = h
    c_ref[...] = c

    @pl.when(t == TP - 1)
    def _finish():
        x = x_ref[...]
        preds = jnp.dot(x, WpT_ref[...],
                        preferred_element_type=jnp.float32)
        preds_ref[...] = preds
        hx_fc = jnp.concatenate([h, x], axis=-1)
        out = jnp.dot(hx_fc, WfcT_ref[...],
                      preferred_element_type=jnp.float32) + bfc_ref[...]
        logits = (out + gn_ref[...]) * (1.0 / TAU)
        m = jnp.max(logits, axis=-1, keepdims=True)
        e = jnp.exp(logits - m)
        probs = e / jnp.sum(e, axis=-1, keepdims=True)
        final_ref[...] = jnp.sum(preds * probs, axis=-1, keepdims=True)


@jax.jit
def kernel(x, hist_loss, Wp, bp, W_ih, W_hh, b_ih, b_hh, Wfc, bfc):
    xs = jnp.swapaxes(hist_loss[:, :TP], 0, 1)
    Wcat = jnp.concatenate([W_ih, W_hh], axis=1).T
    b = (b_ih + b_hh)[None, :]
    WpT = Wp.T
    WfcT = Wfc.T
    gnoise = jax.random.gumbel(jax.random.key(42), (B, S), dtype=jnp.float32)

    grid = (TP,)
    final_pred, preds = pl.pallas_call(
        _lstm_router_kernel,
        grid=grid,
        in_specs=[
            pl.BlockSpec((1, B, S), lambda t: (t, 0, 0)),
            pl.BlockSpec((B, D), lambda t: (0, 0)),
            pl.BlockSpec((S + H, 4 * H), lambda t: (0, 0)),
            pl.BlockSpec((1, 4 * H), lambda t: (0, 0)),
            pl.BlockSpec((D, S), lambda t: (0, 0)),
            pl.BlockSpec((H + D, S), lambda t: (0, 0)),
            pl.BlockSpec((1, S), lambda t: (0, 0)),
            pl.BlockSpec((B, S), lambda t: (0, 0)),
        ],
        out_specs=[
            pl.BlockSpec((B, 1), lambda t: (0, 0)),
            pl.BlockSpec((B, S), lambda t: (0, 0)),
        ],
        out_shape=[
            jax.ShapeDtypeStruct((B, 1), jnp.float32),
            jax.ShapeDtypeStruct((B, S), jnp.float32),
        ],
        scratch_shapes=[
            pltpu.VMEM((B, H), jnp.float32),
            pltpu.VMEM((B, H), jnp.float32),
            pltpu.VMEM((B, S + H), jnp.float32),
        ],
    )(xs, x, Wcat, b, WpT, WfcT, bfc[None, :], gnoise)
    return (final_pred, preds)


# speedup vs baseline: 1.2576x; 1.2221x over previous
import jax
import jax.numpy as jnp
from jax.experimental import pallas as pl
from jax.experimental.pallas import tpu as pltpu

B, D, S, T, H, HOR = 4096, 256, 16, 220, 64, 20
TP = T - HOR
TAU = 1.0
B2 = B // 2
K = 2 * H + 2 * S
G = 8 * H


def _fold(a):
    f = a.shape[-1]
    return jnp.swapaxes(a.reshape(2, B2, f), 0, 1).reshape(B2, 2 * f)


def _unfold(a2):
    f2 = a2.shape[-1]
    return jnp.swapaxes(a2.reshape(B2, 2, f2 // 2), 0, 1).reshape(B, f2 // 2)


def _lstm_router_kernel(xs_ref, xf_ref, Wg_ref, bg_ref, Wp2_ref, Wfh2_ref,
                        Wfx2_ref, bfc2_ref, gn2_ref, final_ref, preds_ref,
                        hx_ref, c_ref):
    t = pl.program_id(0)

    @pl.when(t == 0)
    def _init():
        hx_ref[...] = jnp.zeros_like(hx_ref)
        c_ref[...] = jnp.zeros_like(c_ref)

    hx_ref[:, 2 * H:] = xs_ref[0]
    gates = jnp.dot(hx_ref[...], Wg_ref[...],
                    preferred_element_type=jnp.float32) + bg_ref[...]
    i = jax.nn.sigmoid(gates[:, 0 * 2 * H:1 * 2 * H])
    f = jax.nn.sigmoid(gates[:, 1 * 2 * H:2 * 2 * H])
    g = jnp.tanh(gates[:, 2 * 2 * H:3 * 2 * H])
    o = jax.nn.sigmoid(gates[:, 3 * 2 * H:4 * 2 * H])
    c = f * c_ref[...] + i * g
    h = o * jnp.tanh(c)
    c_ref[...] = c
    hx_ref[:, :2 * H] = h

    @pl.when(t == TP - 1)
    def _finish():
        xf = xf_ref[...]
        preds2 = jnp.dot(xf, Wp2_ref[...],
                         preferred_element_type=jnp.float32)
        preds_ref[...] = preds2
        out2 = (jnp.dot(h, Wfh2_ref[...], preferred_element_type=jnp.float32)
                + jnp.dot(xf, Wfx2_ref[...],
                          preferred_element_type=jnp.float32)
                + bfc2_ref[...])
        logits2 = (out2 + gn2_ref[...]) * (1.0 / TAU)
        lo, hi = logits2[:, :S], logits2[:, S:]
        plo, phi = preds2[:, :S], preds2[:, S:]
        elo = jnp.exp(lo - jnp.max(lo, axis=-1, keepdims=True))
        ehi = jnp.exp(hi - jnp.max(hi, axis=-1, keepdims=True))
        flo = jnp.sum(plo * elo, axis=-1, keepdims=True) / jnp.sum(
            elo, axis=-1, keepdims=True)
        fhi = jnp.sum(phi * ehi, axis=-1, keepdims=True) / jnp.sum(
            ehi, axis=-1, keepdims=True)
        final_ref[...] = jnp.concatenate([flo, fhi], axis=-1)


def _block_diag2(w):
    r, c = w.shape
    z = jnp.zeros((r, c), w.dtype)
    return jnp.block([[w, z], [z, w]])


@jax.jit
def kernel(x, hist_loss, Wp, bp, W_ih, W_hh, b_ih, b_hh, Wfc, bfc):
    xs = jnp.swapaxes(hist_loss[:, :TP], 0, 1)
    xs2 = jnp.swapaxes(xs.reshape(TP, 2, B2, S), 1, 2).reshape(TP, B2, 2 * S)

    WhT = W_hh.T
    WxT = W_ih.T
    b = b_ih + b_hh
    Wg = jnp.zeros((K, G), jnp.float32)
    bg = jnp.zeros((G,), jnp.float32)
    for q in range(4):
        wh = WhT[:, q * H:(q + 1) * H]
        wx = WxT[:, q * H:(q + 1) * H]
        Wg = Wg.at[0:H, q * 2 * H:q * 2 * H + H].set(wh)
        Wg = Wg.at[H:2 * H, q * 2 * H + H:(q + 1) * 2 * H].set(wh)
        Wg = Wg.at[2 * H:2 * H + S, q * 2 * H:q * 2 * H + H].set(wx)
        Wg = Wg.at[2 * H + S:K, q * 2 * H + H:(q + 1) * 2 * H].set(wx)
        bg = bg.at[q * 2 * H:q * 2 * H + H].set(b[q * H:(q + 1) * H])
        bg = bg.at[q * 2 * H + H:(q + 1) * 2 * H].set(b[q * H:(q + 1) * H])

    xf = _fold(x)
    Wp2 = _block_diag2(Wp.T)
    Wfh2 = _block_diag2(Wfc[:, :H].T)
    Wfx2 = _block_diag2(Wfc[:, H:].T)
    bfc2 = jnp.tile(bfc, 2)[None, :]
    gn2 = _fold(jax.random.gumbel(jax.random.key(42), (B, S),
                                  dtype=jnp.float32))

    final2, preds2 = pl.pallas_call(
        _lstm_router_kernel,
        grid=(TP,),
        in_specs=[
            pl.BlockSpec((1, B2, 2 * S), lambda t: (t, 0, 0)),
            pl.BlockSpec((B2, 2 * D), lambda t: (0, 0)),
            pl.BlockSpec((K, G), lambda t: (0, 0)),
            pl.BlockSpec((1, G), lambda t: (0, 0)),
            pl.BlockSpec((2 * D, 2 * S), lambda t: (0, 0)),
            pl.BlockSpec((2 * H, 2 * S), lambda t: (0, 0)),
            pl.BlockSpec((2 * D, 2 * S), lambda t: (0, 0)),
            pl.BlockSpec((1, 2 * S), lambda t: (0, 0)),
            pl.BlockSpec((B2, 2 * S), lambda t: (0, 0)),
        ],
        out_specs=[
            pl.BlockSpec((B2, 2), lambda t: (0, 0)),
            pl.BlockSpec((B2, 2 * S), lambda t: (0, 0)),
        ],
        out_shape=[
            jax.ShapeDtypeStruct((B2, 2), jnp.float32),
            jax.ShapeDtypeStruct((B2, 2 * S), jnp.float32),
        ],
        scratch_shapes=[
            pltpu.VMEM((B2, K), jnp.float32),
            pltpu.VMEM((B2, 2 * H), jnp.float32),
        ],
    )(xs2, xf, Wg, bg[None, :], Wp2, Wfh2, Wfx2, bfc2, gn2)

    final_pred = jnp.swapaxes(final2, 0, 1).reshape(B, 1)
    preds = _unfold(preds2)
    return (final_pred, preds)


# speedup vs baseline: 1.4333x; 1.1397x over previous
import jax
import jax.numpy as jnp
from jax.experimental import pallas as pl
from jax.experimental.pallas import tpu as pltpu

B, D, S, T, H, HOR = 4096, 256, 16, 220, 64, 20
TP = T - HOR
TAU = 1.0
B2 = B // 2
K = 2 * H + 2 * S
G = 8 * H


def _fold(a):
    f = a.shape[-1]
    return jnp.swapaxes(a.reshape(2, B2, f), 0, 1).reshape(B2, 2 * f)


def _unfold(a2):
    f2 = a2.shape[-1]
    return jnp.swapaxes(a2.reshape(B2, 2, f2 // 2), 0, 1).reshape(B, f2 // 2)


BQ = B2 // 2


def _lstm_router_kernel(xs_ref, xf_ref, Wg_ref, bg_ref, Wp2_ref, Wfh2_ref,
                        Wfx2_ref, bfc2_ref, gn2_ref, final_ref, preds_ref,
                        hx_ref, c_ref):
    t = pl.program_id(0)

    @pl.when(t == 0)
    def _init():
        hx_ref[...] = jnp.zeros_like(hx_ref)
        c_ref[...] = jnp.zeros_like(c_ref)

    def step(r):
        rows = pl.ds(r * BQ, BQ)
        hx_ref[rows, 2 * H:] = xs_ref[0, rows]
        gates = jnp.dot(hx_ref[rows, :], Wg_ref[...],
                        preferred_element_type=jnp.float32) + bg_ref[...]
        i = jnp.tanh(gates[:, 0 * 2 * H:1 * 2 * H]) * 0.5 + 0.5
        f = jnp.tanh(gates[:, 1 * 2 * H:2 * 2 * H]) * 0.5 + 0.5
        g = jnp.tanh(gates[:, 2 * 2 * H:3 * 2 * H])
        o = jnp.tanh(gates[:, 3 * 2 * H:4 * 2 * H]) * 0.5 + 0.5
        c = f * c_ref[rows, :] + i * g
        h = o * jnp.tanh(c)
        c_ref[rows, :] = c
        hx_ref[rows, :2 * H] = h
        return h

    hA = step(0)
    hB = step(1)

    @pl.when(t == TP - 1)
    def _finish():
        for r, h in ((0, hA), (1, hB)):
            rows = pl.ds(r * BQ, BQ)
            xf = xf_ref[rows, :]
            preds2 = jnp.dot(xf, Wp2_ref[...],
                             preferred_element_type=jnp.float32)
            preds_ref[rows, :] = preds2
            out2 = (jnp.dot(h, Wfh2_ref[...],
                            preferred_element_type=jnp.float32)
                    + jnp.dot(xf, Wfx2_ref[...],
                              preferred_element_type=jnp.float32)
                    + bfc2_ref[...])
            logits2 = (out2 + gn2_ref[rows, :]) * (1.0 / TAU)
            lo, hi = logits2[:, :S], logits2[:, S:]
            plo, phi = preds2[:, :S], preds2[:, S:]
            elo = jnp.exp(lo - jnp.max(lo, axis=-1, keepdims=True))
            ehi = jnp.exp(hi - jnp.max(hi, axis=-1, keepdims=True))
            flo = jnp.sum(plo * elo, axis=-1, keepdims=True) / jnp.sum(
                elo, axis=-1, keepdims=True)
            fhi = jnp.sum(phi * ehi, axis=-1, keepdims=True) / jnp.sum(
                ehi, axis=-1, keepdims=True)
            final_ref[rows, :] = jnp.concatenate([flo, fhi], axis=-1)


def _block_diag2(w):
    r, c = w.shape
    z = jnp.zeros((r, c), w.dtype)
    return jnp.block([[w, z], [z, w]])


@jax.jit
def kernel(x, hist_loss, Wp, bp, W_ih, W_hh, b_ih, b_hh, Wfc, bfc):
    xs = jnp.swapaxes(hist_loss[:, :TP], 0, 1)
    xs2 = jnp.swapaxes(xs.reshape(TP, 2, B2, S), 1, 2).reshape(TP, B2, 2 * S)

    WhT = W_hh.T
    WxT = W_ih.T
    b = b_ih + b_hh
    Wg = jnp.zeros((K, G), jnp.float32)
    bg = jnp.zeros((G,), jnp.float32)
    for q in range(4):
        s = 1.0 if q == 2 else 0.5
        wh = WhT[:, q * H:(q + 1) * H] * s
        wx = WxT[:, q * H:(q + 1) * H] * s
        Wg = Wg.at[0:H, q * 2 * H:q * 2 * H + H].set(wh)
        Wg = Wg.at[H:2 * H, q * 2 * H + H:(q + 1) * 2 * H].set(wh)
        Wg = Wg.at[2 * H:2 * H + S, q * 2 * H:q * 2 * H + H].set(wx)
        Wg = Wg.at[2 * H + S:K, q * 2 * H + H:(q + 1) * 2 * H].set(wx)
        bg = bg.at[q * 2 * H:q * 2 * H + H].set(b[q * H:(q + 1) * H] * s)
        bg = bg.at[q * 2 * H + H:(q + 1) * 2 * H].set(b[q * H:(q + 1) * H] * s)

    xf = _fold(x)
    Wp2 = _block_diag2(Wp.T)
    Wfh2 = _block_diag2(Wfc[:, :H].T)
    Wfx2 = _block_diag2(Wfc[:, H:].T)
    bfc2 = jnp.tile(bfc, 2)[None, :]
    gn2 = _fold(jax.random.gumbel(jax.random.key(42), (B, S),
                                  dtype=jnp.float32))

    final2, preds2 = pl.pallas_call(
        _lstm_router_kernel,
        grid=(TP,),
        in_specs=[
            pl.BlockSpec((1, B2, 2 * S), lambda t: (t, 0, 0)),
            pl.BlockSpec((B2, 2 * D), lambda t: (0, 0)),
            pl.BlockSpec((K, G), lambda t: (0, 0)),
            pl.BlockSpec((1, G), lambda t: (0, 0)),
            pl.BlockSpec((2 * D, 2 * S), lambda t: (0, 0)),
            pl.BlockSpec((2 * H, 2 * S), lambda t: (0, 0)),
            pl.BlockSpec((2 * D, 2 * S), lambda t: (0, 0)),
            pl.BlockSpec((1, 2 * S), lambda t: (0, 0)),
            pl.BlockSpec((B2, 2 * S), lambda t: (0, 0)),
        ],
        out_specs=[
            pl.BlockSpec((B2, 2), lambda t: (0, 0)),
            pl.BlockSpec((B2, 2 * S), lambda t: (0, 0)),
        ],
        out_shape=[
            jax.ShapeDtypeStruct((B2, 2), jnp.float32),
            jax.ShapeDtypeStruct((B2, 2 * S), jnp.float32),
        ],
        scratch_shapes=[
            pltpu.VMEM((B2, K), jnp.float32),
            pltpu.VMEM((B2, 2 * H), jnp.float32),
        ],
    )(xs2, xf, Wg, bg[None, :], Wp2, Wfh2, Wfx2, bfc2, gn2)

    final_pred = jnp.swapaxes(final2, 0, 1).reshape(B, 1)
    preds = _unfold(preds2)
    return (final_pred, preds)
